# baseline (speedup 1.0000x reference)
import functools

import jax
import jax.numpy as jnp
from jax import lax
from jax.experimental import pallas as pl
from jax.experimental.pallas import tpu as pltpu
from jax.experimental.pallas import tpu_sc as plsc

T = 2048
D = 1024
E = 8
F = 4096
TOPK = 2

F_TILE = 1024
NF = F // F_TILE
TB = 256
NB = 24
PADN = NB * TB

NW = 32
CH = T // NW
SUB = 32


def _router_body(x_ref, wr_ref, pos0_ref, pos1_ref, w0_ref, w1_ref,
                 bexp_ref, bvalid_ref, xbf_ref):
    xbf_ref[...] = x_ref[...].astype(jnp.bfloat16)
    logits = jnp.dot(x_ref[...], wr_ref[...], preferred_element_type=jnp.float32)
    eidx = lax.broadcasted_iota(jnp.int32, (T, E), 1)
    m1 = jnp.max(logits, axis=-1, keepdims=True)
    e0 = jnp.min(jnp.where(logits == m1, eidx, E), axis=-1, keepdims=True)
    l2 = jnp.where(eidx == e0, -jnp.inf, logits)
    m2 = jnp.max(l2, axis=-1, keepdims=True)
    e1 = jnp.min(jnp.where(l2 == m2, eidx, E), axis=-1, keepdims=True)
    w0 = 1.0 / (1.0 + jnp.exp(m2 - m1))
    w0_ref[...] = w0
    w1_ref[...] = 1.0 - w0

    oh0 = (eidx == e0).astype(jnp.float32)
    oh1 = (eidx == e1).astype(jnp.float32)

    CB = 256
    r = lax.broadcasted_iota(jnp.int32, (CB, CB), 0)
    c = lax.broadcasted_iota(jnp.int32, (CB, CB), 1)
    ltri = (r >= c).astype(jnp.float32)

    def ranks(oh):
        tot = jnp.zeros((1, E), jnp.float32)
        parts = []
        for b in range(T // CB):
            blk = oh[b * CB:(b + 1) * CB, :]
            cum = jnp.dot(ltri, blk, preferred_element_type=jnp.float32) + tot
            parts.append(jnp.sum(cum * blk, axis=1, keepdims=True) - 1.0)
            tot = tot + jnp.sum(blk, axis=0, keepdims=True)
        return jnp.concatenate(parts, axis=0), tot

    rank0, cnt0 = ranks(oh0)
    rank1, cnt1 = ranks(oh1)
    cnt = cnt0 + cnt1
    nblk = jnp.floor((cnt + (TB - 1)) / TB)
    stri = (lax.broadcasted_iota(jnp.int32, (E, E), 0)
            < lax.broadcasted_iota(jnp.int32, (E, E), 1)).astype(jnp.float32)
    offblk = jnp.dot(nblk, stri, preferred_element_type=jnp.float32)
    off = offblk * TB

    off0 = jnp.sum(oh0 * off, axis=1, keepdims=True)
    off1 = jnp.sum(oh1 * off, axis=1, keepdims=True)
    c0at1 = jnp.sum(oh1 * cnt0, axis=1, keepdims=True)
    pos0_ref[...] = (off0 + rank0).astype(jnp.int32)
    pos1_ref[...] = (off1 + c0at1 + rank1).astype(jnp.int32)

    bidx = lax.broadcasted_iota(jnp.int32, (NB, E), 0).astype(jnp.float32)
    bexp = jnp.sum((bidx >= offblk).astype(jnp.float32), axis=1,
                   keepdims=True) - 1.0
    bexp = jnp.clip(bexp, 0.0, E - 1.0)
    ohb = (lax.broadcasted_iota(jnp.int32, (NB, E), 1).astype(jnp.float32)
           == bexp)
    offb = jnp.sum(jnp.where(ohb, off, 0.0), axis=1, keepdims=True)
    cntb = jnp.sum(jnp.where(ohb, cnt, 0.0), axis=1, keepdims=True)
    brow = lax.broadcasted_iota(jnp.int32, (NB, 1), 0).astype(jnp.float32) * TB
    bexp_ref[...] = bexp.astype(jnp.int32)
    bvalid_ref[...] = ((brow - offb) < cntb).astype(jnp.int32)


def _router_call():
    return pl.pallas_call(
        _router_body,
        out_shape=(
            jax.ShapeDtypeStruct((T, 1), jnp.int32),
            jax.ShapeDtypeStruct((T, 1), jnp.int32),
            jax.ShapeDtypeStruct((T, 1), jnp.float32),
            jax.ShapeDtypeStruct((T, 1), jnp.float32),
            jax.ShapeDtypeStruct((NB, 1), jnp.int32),
            jax.ShapeDtypeStruct((NB, 1), jnp.int32),
            jax.ShapeDtypeStruct((T, D), jnp.bfloat16),
        ),
    )


@functools.lru_cache(maxsize=None)
def _make_sc_scatter():
    mesh = plsc.VectorSubcoreMesh(core_axis_name="c", subcore_axis_name="s")

    @functools.partial(
        pl.kernel,
        mesh=mesh,
        out_type=jax.ShapeDtypeStruct((PADN, D // 2), jnp.int32),
        scratch_types=[
            pltpu.VMEM((CH, D // 2), jnp.int32),
            pltpu.VMEM((CH,), jnp.int32),
            pltpu.VMEM((CH,), jnp.int32),
            pltpu.SemaphoreType.DMA,
            pltpu.SemaphoreType.DMA,
        ],
    )
    def _sc_scatter(x_hbm, pos0_hbm, pos1_hbm, xs_hbm, rows_v, idx0_v, idx1_v,
                    sem0, sem1):
        wid = lax.axis_index("s") * 2 + lax.axis_index("c")
        base = wid * CH
        pltpu.sync_copy(x_hbm.at[pl.ds(base, CH)], rows_v)
        pltpu.sync_copy(pos0_hbm.at[pl.ds(base, CH)], idx0_v)
        pltpu.sync_copy(pos1_hbm.at[pl.ds(base, CH)], idx1_v)
        cp0 = pltpu.async_copy(rows_v, xs_hbm.at[idx0_v], sem0)
        cp1 = pltpu.async_copy(rows_v, xs_hbm.at[idx1_v], sem1)
        cp0.wait()
        cp1.wait()

    return _sc_scatter


def _gmm_body(bexp_sm, bvalid_sm, xs_ref, wg_ref, wu_ref, wd_ref, out_ref,
              acc_ref):
    f = pl.program_id(0)
    b = pl.program_id(1)

    @pl.when(bvalid_sm[b] == 1)
    def _():
        xb = xs_ref[...]
        g = jnp.dot(xb, wg_ref[0].astype(jnp.bfloat16),
                    preferred_element_type=jnp.float32)
        u = jnp.dot(xb, wu_ref[0].astype(jnp.bfloat16),
                    preferred_element_type=jnp.float32)
        h = (g * jax.nn.sigmoid(g)) * u
        y = jnp.dot(h.astype(jnp.bfloat16), wd_ref[0].astype(jnp.bfloat16),
                    preferred_element_type=jnp.float32)

        @pl.when(f == 0)
        def _():
            acc_ref[pl.ds(b * TB, TB), :] = y

        @pl.when(f != 0)
        def _():
            acc_ref[pl.ds(b * TB, TB), :] += y

    @pl.when(f == NF - 1)
    def _():
        out_ref[...] = acc_ref[pl.ds(b * TB, TB), :]


def _gmm_call():
    return pl.pallas_call(
        _gmm_body,
        grid_spec=pltpu.PrefetchScalarGridSpec(
            num_scalar_prefetch=2,
            grid=(NF, NB),
            in_specs=[
                pl.BlockSpec((TB, D), lambda f, b, be, bv: (b, 0)),
                pl.BlockSpec((1, D, F_TILE),
                             lambda f, b, be, bv: (be[b], 0, f)),
                pl.BlockSpec((1, D, F_TILE),
                             lambda f, b, be, bv: (be[b], 0, f)),
                pl.BlockSpec((1, F_TILE, D),
                             lambda f, b, be, bv: (be[b], f, 0)),
            ],
            out_specs=pl.BlockSpec(
                (TB, D),
                lambda f, b, be, bv: (jnp.where(f == NF - 1, b, 0), 0)),
            scratch_shapes=[pltpu.VMEM((PADN, D), jnp.float32)],
        ),
        out_shape=jax.ShapeDtypeStruct((PADN, D), jnp.float32),
    )


@functools.lru_cache(maxsize=None)
def _make_sc_combine():
    mesh = plsc.VectorSubcoreMesh(core_axis_name="c", subcore_axis_name="s")

    @functools.partial(
        pl.kernel,
        mesh=mesh,
        out_type=jax.ShapeDtypeStruct((T, D), jnp.float32),
        scratch_types=[
            pltpu.VMEM((SUB, D), jnp.float32),
            pltpu.VMEM((SUB, D), jnp.float32),
            pltpu.VMEM((SUB,), jnp.int32),
            pltpu.VMEM((SUB,), jnp.int32),
            pltpu.VMEM((CH,), jnp.float32),
            pltpu.VMEM((CH,), jnp.float32),
            pltpu.SemaphoreType.DMA,
            pltpu.SemaphoreType.DMA,
        ],
    )
    def _sc_combine(ys_hbm, pos0_hbm, pos1_hbm, w0_hbm, w1_hbm, out_hbm,
                    g0_v, g1_v, i0_v, i1_v, w0_v, w1_v, sem0, sem1):
        wid = lax.axis_index("s") * 2 + lax.axis_index("c")
        base = wid * CH
        pltpu.sync_copy(w0_hbm.at[pl.ds(base, CH)], w0_v)
        pltpu.sync_copy(w1_hbm.at[pl.ds(base, CH)], w1_v)
        for sc in range(CH // SUB):
            sbase = base + sc * SUB
            pltpu.sync_copy(pos0_hbm.at[pl.ds(sbase, SUB)], i0_v)
            pltpu.sync_copy(pos1_hbm.at[pl.ds(sbase, SUB)], i1_v)
            cp0 = pltpu.async_copy(ys_hbm.at[i0_v], g0_v, sem0)
            cp1 = pltpu.async_copy(ys_hbm.at[i1_v], g1_v, sem1)
            cp0.wait()
            cp1.wait()
            for r in range(SUB):
                lane = (sc * SUB + r) % 16
                chunk = (sc * SUB + r) - lane
                a = w0_v[pl.ds(chunk, 16)][lane]
                bw = w1_v[pl.ds(chunk, 16)][lane]

                def col(j, _):
                    v0 = g0_v[r, pl.ds(j * 16, 16)]
                    v1 = g1_v[r, pl.ds(j * 16, 16)]
                    g0_v[r, pl.ds(j * 16, 16)] = v0 * a + v1 * bw
                    return _

                lax.fori_loop(0, D // 16, col, 0)
            pltpu.sync_copy(g0_v, out_hbm.at[pl.ds(sbase, SUB)])

    return _sc_combine


def kernel(x, W_router, W_gate, W_up, W_down):
    pos0, pos1, w0, w1, bexp, bvalid, xbf = _router_call()(x, W_router)
    pos0 = pos0.reshape(T)
    pos1 = pos1.reshape(T)
    x_i32 = jax.lax.bitcast_convert_type(
        xbf.reshape(T, D // 2, 2), jnp.int32)
    xs_p = _make_sc_scatter()(x_i32, pos0, pos1)
    xs = jax.lax.bitcast_convert_type(xs_p, jnp.bfloat16).reshape(PADN, D)
    ys = _gmm_call()(bexp.reshape(NB), bvalid.reshape(NB), xs, W_gate, W_up,
                     W_down)
    out = _make_sc_combine()(ys, pos0, pos1, w0.reshape(T), w1.reshape(T))
    return out


# speedup vs baseline: 1.4717x; 1.4717x over previous
import functools

import jax
import jax.numpy as jnp
from jax import lax
from jax.experimental import pallas as pl
from jax.experimental.pallas import tpu as pltpu
from jax.experimental.pallas import tpu_sc as plsc

T = 2048
D = 1024
E = 8
F = 4096
TOPK = 2

F_TILE = 1024
NF = F // F_TILE
TB = 256
NB = 24
PADN = NB * TB

NW = 32
CH = T // NW
SUB = 32


def _router_body(x_ref, wr_ref, pos0_ref, pos1_ref, w0_ref, w1_ref,
                 bexp_ref, bvalid_ref):
    logits = jnp.dot(x_ref[...], wr_ref[...], preferred_element_type=jnp.float32)
    eidx = lax.broadcasted_iota(jnp.int32, (T, E), 1)
    m1 = jnp.max(logits, axis=-1, keepdims=True)
    e0 = jnp.min(jnp.where(logits == m1, eidx, E), axis=-1, keepdims=True)
    l2 = jnp.where(eidx == e0, -jnp.inf, logits)
    m2 = jnp.max(l2, axis=-1, keepdims=True)
    e1 = jnp.min(jnp.where(l2 == m2, eidx, E), axis=-1, keepdims=True)
    w0 = 1.0 / (1.0 + jnp.exp(m2 - m1))
    w0_ref[...] = w0
    w1_ref[...] = 1.0 - w0

    oh0 = (eidx == e0).astype(jnp.float32)
    oh1 = (eidx == e1).astype(jnp.float32)

    CB = 256
    r = lax.broadcasted_iota(jnp.int32, (CB, CB), 0)
    c = lax.broadcasted_iota(jnp.int32, (CB, CB), 1)
    ltri = (r >= c).astype(jnp.float32)

    def ranks(oh):
        tot = jnp.zeros((1, E), jnp.float32)
        parts = []
        for b in range(T // CB):
            blk = oh[b * CB:(b + 1) * CB, :]
            cum = jnp.dot(ltri, blk, preferred_element_type=jnp.float32) + tot
            parts.append(jnp.sum(cum * blk, axis=1, keepdims=True) - 1.0)
            tot = tot + jnp.sum(blk, axis=0, keepdims=True)
        return jnp.concatenate(parts, axis=0), tot

    rank0, cnt0 = ranks(oh0)
    rank1, cnt1 = ranks(oh1)
    cnt = cnt0 + cnt1
    nblk = jnp.floor((cnt + (TB - 1)) / TB)
    stri = (lax.broadcasted_iota(jnp.int32, (E, E), 0)
            < lax.broadcasted_iota(jnp.int32, (E, E), 1)).astype(jnp.float32)
    offblk = jnp.dot(nblk, stri, preferred_element_type=jnp.float32)
    off = offblk * TB

    off0 = jnp.sum(oh0 * off, axis=1, keepdims=True)
    off1 = jnp.sum(oh1 * off, axis=1, keepdims=True)
    c0at1 = jnp.sum(oh1 * cnt0, axis=1, keepdims=True)
    pos0_ref[...] = (off0 + rank0).astype(jnp.int32)
    pos1_ref[...] = (off1 + c0at1 + rank1).astype(jnp.int32)

    bidx = lax.broadcasted_iota(jnp.int32, (NB, E), 0).astype(jnp.float32)
    bexp = jnp.sum((bidx >= offblk).astype(jnp.float32), axis=1,
                   keepdims=True) - 1.0
    bexp = jnp.clip(bexp, 0.0, E - 1.0)
    ohb = (lax.broadcasted_iota(jnp.int32, (NB, E), 1).astype(jnp.float32)
           == bexp)
    offb = jnp.sum(jnp.where(ohb, off, 0.0), axis=1, keepdims=True)
    cntb = jnp.sum(jnp.where(ohb, cnt, 0.0), axis=1, keepdims=True)
    brow = lax.broadcasted_iota(jnp.int32, (NB, 1), 0).astype(jnp.float32) * TB
    bexp_ref[...] = bexp.astype(jnp.int32)
    bvalid_ref[...] = ((brow - offb) < cntb).astype(jnp.int32)


def _router_call():
    return pl.pallas_call(
        _router_body,
        out_shape=(
            jax.ShapeDtypeStruct((T, 1), jnp.int32),
            jax.ShapeDtypeStruct((T, 1), jnp.int32),
            jax.ShapeDtypeStruct((T, 1), jnp.float32),
            jax.ShapeDtypeStruct((T, 1), jnp.float32),
            jax.ShapeDtypeStruct((NB, 1), jnp.int32),
            jax.ShapeDtypeStruct((NB, 1), jnp.int32),
        ),
    )


@functools.lru_cache(maxsize=None)
def _make_sc_scatter():
    mesh = plsc.VectorSubcoreMesh(core_axis_name="c", subcore_axis_name="s")

    @functools.partial(
        pl.kernel,
        mesh=mesh,
        out_type=jax.ShapeDtypeStruct((PADN, D), jnp.float32),
        scratch_types=[
            pltpu.VMEM((CH, D), jnp.float32),
            pltpu.VMEM((CH,), jnp.int32),
            pltpu.VMEM((CH,), jnp.int32),
            pltpu.SemaphoreType.DMA,
            pltpu.SemaphoreType.DMA,
        ],
    )
    def _sc_scatter(x_hbm, pos0_hbm, pos1_hbm, xs_hbm, rows_v, idx0_v, idx1_v,
                    sem0, sem1):
        wid = lax.axis_index("s") * 2 + lax.axis_index("c")
        base = wid * CH
        pltpu.sync_copy(x_hbm.at[pl.ds(base, CH)], rows_v)
        pltpu.sync_copy(pos0_hbm.at[pl.ds(base, CH)], idx0_v)
        pltpu.sync_copy(pos1_hbm.at[pl.ds(base, CH)], idx1_v)
        cp0 = pltpu.async_copy(rows_v, xs_hbm.at[idx0_v], sem0)
        cp1 = pltpu.async_copy(rows_v, xs_hbm.at[idx1_v], sem1)
        cp0.wait()
        cp1.wait()

    return _sc_scatter


def _gmm_body(bexp_sm, bvalid_sm, xs_ref, wg_ref, wu_ref, wd_ref, out_ref,
              acc_ref):
    f = pl.program_id(0)
    b = pl.program_id(1)

    @pl.when(bvalid_sm[b] == 1)
    def _():
        xb = xs_ref[...].astype(jnp.bfloat16)
        g = jnp.dot(xb, wg_ref[0].astype(jnp.bfloat16),
                    preferred_element_type=jnp.float32)
        u = jnp.dot(xb, wu_ref[0].astype(jnp.bfloat16),
                    preferred_element_type=jnp.float32)
        h = (g * jax.nn.sigmoid(g)) * u
        y = jnp.dot(h.astype(jnp.bfloat16), wd_ref[0].astype(jnp.bfloat16),
                    preferred_element_type=jnp.float32)

        @pl.when(f == 0)
        def _():
            acc_ref[pl.ds(b * TB, TB), :] = y

        @pl.when(f != 0)
        def _():
            acc_ref[pl.ds(b * TB, TB), :] += y

    @pl.when(f == NF - 1)
    def _():
        out_ref[...] = acc_ref[pl.ds(b * TB, TB), :]


def _gmm_call():
    return pl.pallas_call(
        _gmm_body,
        grid_spec=pltpu.PrefetchScalarGridSpec(
            num_scalar_prefetch=2,
            grid=(NF, NB),
            in_specs=[
                pl.BlockSpec((TB, D), lambda f, b, be, bv: (b, 0)),
                pl.BlockSpec((1, D, F_TILE),
                             lambda f, b, be, bv: (be[b], 0, f)),
                pl.BlockSpec((1, D, F_TILE),
                             lambda f, b, be, bv: (be[b], 0, f)),
                pl.BlockSpec((1, F_TILE, D),
                             lambda f, b, be, bv: (be[b], f, 0)),
            ],
            out_specs=pl.BlockSpec(
                (TB, D),
                lambda f, b, be, bv: (jnp.where(f == NF - 1, b, 0), 0)),
            scratch_shapes=[pltpu.VMEM((PADN, D), jnp.float32)],
        ),
        out_shape=jax.ShapeDtypeStruct((PADN, D), jnp.float32),
    )


@functools.lru_cache(maxsize=None)
def _make_sc_combine():
    mesh = plsc.VectorSubcoreMesh(core_axis_name="c", subcore_axis_name="s")

    @functools.partial(
        pl.kernel,
        mesh=mesh,
        out_type=jax.ShapeDtypeStruct((T, D), jnp.float32),
        scratch_types=[
            pltpu.VMEM((SUB, D), jnp.float32),
            pltpu.VMEM((SUB, D), jnp.float32),
            pltpu.VMEM((SUB,), jnp.int32),
            pltpu.VMEM((SUB,), jnp.int32),
            pltpu.VMEM((CH,), jnp.float32),
            pltpu.VMEM((CH,), jnp.float32),
            pltpu.SemaphoreType.DMA,
            pltpu.SemaphoreType.DMA,
        ],
    )
    def _sc_combine(ys_hbm, pos0_hbm, pos1_hbm, w0_hbm, w1_hbm, out_hbm,
                    g0_v, g1_v, i0_v, i1_v, w0_v, w1_v, sem0, sem1):
        wid = lax.axis_index("s") * 2 + lax.axis_index("c")
        base = wid * CH
        pltpu.sync_copy(w0_hbm.at[pl.ds(base, CH)], w0_v)
        pltpu.sync_copy(w1_hbm.at[pl.ds(base, CH)], w1_v)
        for sc in range(CH // SUB):
            sbase = base + sc * SUB
            pltpu.sync_copy(pos0_hbm.at[pl.ds(sbase, SUB)], i0_v)
            pltpu.sync_copy(pos1_hbm.at[pl.ds(sbase, SUB)], i1_v)
            cp0 = pltpu.async_copy(ys_hbm.at[i0_v], g0_v, sem0)
            cp1 = pltpu.async_copy(ys_hbm.at[i1_v], g1_v, sem1)
            cp0.wait()
            cp1.wait()
            for r in range(SUB):
                lane = (sc * SUB + r) % 16
                chunk = (sc * SUB + r) - lane
                a = w0_v[pl.ds(chunk, 16)][lane]
                bw = w1_v[pl.ds(chunk, 16)][lane]

                def col(j, _):
                    v0 = g0_v[r, pl.ds(j * 16, 16)]
                    v1 = g1_v[r, pl.ds(j * 16, 16)]
                    g0_v[r, pl.ds(j * 16, 16)] = v0 * a + v1 * bw
                    return _

                lax.fori_loop(0, D // 16, col, 0)
            pltpu.sync_copy(g0_v, out_hbm.at[pl.ds(sbase, SUB)])

    return _sc_combine


def kernel(x, W_router, W_gate, W_up, W_down):
    pos0, pos1, w0, w1, bexp, bvalid = _router_call()(x, W_router)
    pos0 = pos0.reshape(T)
    pos1 = pos1.reshape(T)
    xs = _make_sc_scatter()(x, pos0, pos1)
    ys = _gmm_call()(bexp.reshape(NB), bvalid.reshape(NB), xs, W_gate, W_up,
                     W_down)
    out = _make_sc_combine()(ys, pos0, pos1, w0.reshape(T), w1.reshape(T))
    return out


# speedup vs baseline: 1.5227x; 1.0346x over previous
import functools

import jax
import jax.numpy as jnp
from jax import lax
from jax.experimental import pallas as pl
from jax.experimental.pallas import tpu as pltpu
from jax.experimental.pallas import tpu_sc as plsc

T = 2048
D = 1024
E = 8
F = 4096
TOPK = 2

F_TILE = 1024
NF = F // F_TILE
TB = 256
NB = 24
PADN = NB * TB

NW = 32
CH = T // NW
SUB = 32


def _router_body(x_ref, wr_ref, pos0_ref, pos1_ref, w0_ref, w1_ref,
                 bexp_ref, bvalid_ref):
    logits = jnp.dot(x_ref[...], wr_ref[...], preferred_element_type=jnp.float32)
    eidx = lax.broadcasted_iota(jnp.int32, (T, E), 1)
    m1 = jnp.max(logits, axis=-1, keepdims=True)
    e0 = jnp.min(jnp.where(logits == m1, eidx, E), axis=-1, keepdims=True)
    l2 = jnp.where(eidx == e0, -jnp.inf, logits)
    m2 = jnp.max(l2, axis=-1, keepdims=True)
    e1 = jnp.min(jnp.where(l2 == m2, eidx, E), axis=-1, keepdims=True)
    w0 = 1.0 / (1.0 + jnp.exp(m2 - m1))
    w0_ref[...] = w0
    w1_ref[...] = 1.0 - w0

    oh0 = (eidx == e0).astype(jnp.float32)
    oh1 = (eidx == e1).astype(jnp.float32)

    CB = 256
    r = lax.broadcasted_iota(jnp.int32, (CB, CB), 0)
    c = lax.broadcasted_iota(jnp.int32, (CB, CB), 1)
    ltri = (r >= c).astype(jnp.float32)

    def ranks(oh):
        tot = jnp.zeros((1, E), jnp.float32)
        parts = []
        for b in range(T // CB):
            blk = oh[b * CB:(b + 1) * CB, :]
            cum = jnp.dot(ltri, blk, preferred_element_type=jnp.float32) + tot
            parts.append(jnp.sum(cum * blk, axis=1, keepdims=True) - 1.0)
            tot = tot + jnp.sum(blk, axis=0, keepdims=True)
        return jnp.concatenate(parts, axis=0), tot

    rank0, cnt0 = ranks(oh0)
    rank1, cnt1 = ranks(oh1)
    cnt = cnt0 + cnt1
    nblk = jnp.floor((cnt + (TB - 1)) / TB)
    stri = (lax.broadcasted_iota(jnp.int32, (E, E), 0)
            < lax.broadcasted_iota(jnp.int32, (E, E), 1)).astype(jnp.float32)
    offblk = jnp.dot(nblk, stri, preferred_element_type=jnp.float32)
    off = offblk * TB

    off0 = jnp.sum(oh0 * off, axis=1, keepdims=True)
    off1 = jnp.sum(oh1 * off, axis=1, keepdims=True)
    c0at1 = jnp.sum(oh1 * cnt0, axis=1, keepdims=True)
    pos0_ref[...] = (off0 + rank0).astype(jnp.int32)
    pos1_ref[...] = (off1 + c0at1 + rank1).astype(jnp.int32)

    bidx = lax.broadcasted_iota(jnp.int32, (NB, E), 0).astype(jnp.float32)
    bexp = jnp.sum((bidx >= offblk).astype(jnp.float32), axis=1,
                   keepdims=True) - 1.0
    bexp = jnp.clip(bexp, 0.0, E - 1.0)
    ohb = (lax.broadcasted_iota(jnp.int32, (NB, E), 1).astype(jnp.float32)
           == bexp)
    offb = jnp.sum(jnp.where(ohb, off, 0.0), axis=1, keepdims=True)
    cntb = jnp.sum(jnp.where(ohb, cnt, 0.0), axis=1, keepdims=True)
    brow = lax.broadcasted_iota(jnp.int32, (NB, 1), 0).astype(jnp.float32) * TB
    bexp_ref[...] = bexp.astype(jnp.int32)
    bvalid_ref[...] = ((brow - offb) < cntb).astype(jnp.int32)


def _router_call():
    return pl.pallas_call(
        _router_body,
        out_shape=(
            jax.ShapeDtypeStruct((T, 1), jnp.int32),
            jax.ShapeDtypeStruct((T, 1), jnp.int32),
            jax.ShapeDtypeStruct((T, 1), jnp.float32),
            jax.ShapeDtypeStruct((T, 1), jnp.float32),
            jax.ShapeDtypeStruct((NB, 1), jnp.int32),
            jax.ShapeDtypeStruct((NB, 1), jnp.int32),
        ),
    )


@functools.lru_cache(maxsize=None)
def _make_sc_scatter():
    mesh = plsc.VectorSubcoreMesh(core_axis_name="c", subcore_axis_name="s")

    @functools.partial(
        pl.kernel,
        mesh=mesh,
        out_type=jax.ShapeDtypeStruct((PADN, D), jnp.float32),
        scratch_types=[
            pltpu.VMEM((CH, D), jnp.float32),
            pltpu.VMEM((CH,), jnp.int32),
            pltpu.VMEM((CH,), jnp.int32),
            pltpu.SemaphoreType.DMA,
            pltpu.SemaphoreType.DMA,
        ],
    )
    def _sc_scatter(x_hbm, pos0_hbm, pos1_hbm, xs_hbm, rows_v, idx0_v, idx1_v,
                    sem0, sem1):
        wid = lax.axis_index("s") * 2 + lax.axis_index("c")
        base = wid * CH
        pltpu.sync_copy(x_hbm.at[pl.ds(base, CH)], rows_v)
        pltpu.sync_copy(pos0_hbm.at[pl.ds(base, CH)], idx0_v)
        pltpu.sync_copy(pos1_hbm.at[pl.ds(base, CH)], idx1_v)
        cp0 = pltpu.async_copy(rows_v, xs_hbm.at[idx0_v], sem0)
        cp1 = pltpu.async_copy(rows_v, xs_hbm.at[idx1_v], sem1)
        cp0.wait()
        cp1.wait()

    return _sc_scatter


def _gmm_body(bexp_sm, bvalid_sm, xs_ref, wg_ref, wu_ref, wd_ref, out_ref,
              acc_ref):
    f = pl.program_id(0)
    b = pl.program_id(1)

    @pl.when(bvalid_sm[b] == 1)
    def _():
        xb = xs_ref[...].astype(jnp.bfloat16)
        g = jnp.dot(xb, wg_ref[0].astype(jnp.bfloat16),
                    preferred_element_type=jnp.float32)
        u = jnp.dot(xb, wu_ref[0].astype(jnp.bfloat16),
                    preferred_element_type=jnp.float32)
        h = (g * jax.nn.sigmoid(g)) * u
        y = jnp.dot(h.astype(jnp.bfloat16), wd_ref[0].astype(jnp.bfloat16),
                    preferred_element_type=jnp.float32)

        @pl.when(f == 0)
        def _():
            acc_ref[pl.ds(b * TB, TB), :] = y

        @pl.when(f != 0)
        def _():
            acc_ref[pl.ds(b * TB, TB), :] += y

    @pl.when(f == NF - 1)
    def _():
        out_ref[...] = acc_ref[pl.ds(b * TB, TB), :]


def _gmm_call():
    return pl.pallas_call(
        _gmm_body,
        grid_spec=pltpu.PrefetchScalarGridSpec(
            num_scalar_prefetch=2,
            grid=(NF, NB),
            in_specs=[
                pl.BlockSpec((TB, D), lambda f, b, be, bv: (b, 0)),
                pl.BlockSpec((1, D, F_TILE),
                             lambda f, b, be, bv: (be[b], 0, f)),
                pl.BlockSpec((1, D, F_TILE),
                             lambda f, b, be, bv: (be[b], 0, f)),
                pl.BlockSpec((1, F_TILE, D),
                             lambda f, b, be, bv: (be[b], f, 0)),
            ],
            out_specs=pl.BlockSpec(
                (TB, D),
                lambda f, b, be, bv: (jnp.where(f == NF - 1, b, 0), 0)),
            scratch_shapes=[pltpu.VMEM((PADN, D), jnp.float32)],
        ),
        out_shape=jax.ShapeDtypeStruct((PADN, D), jnp.float32),
    )


@functools.lru_cache(maxsize=None)
def _make_sc_combine():
    mesh = plsc.VectorSubcoreMesh(core_axis_name="c", subcore_axis_name="s")

    @functools.partial(
        pl.kernel,
        mesh=mesh,
        out_type=jax.ShapeDtypeStruct((T, D), jnp.float32),
        scratch_types=[
            pltpu.VMEM((SUB, D), jnp.float32),
            pltpu.VMEM((SUB, D), jnp.float32),
            pltpu.VMEM((SUB,), jnp.int32),
            pltpu.VMEM((SUB,), jnp.int32),
            pltpu.VMEM((CH,), jnp.float32),
            pltpu.VMEM((CH,), jnp.float32),
            pltpu.SemaphoreType.DMA,
            pltpu.SemaphoreType.DMA,
        ],
    )
    def _sc_combine(ys_hbm, pos0_hbm, pos1_hbm, w0_hbm, w1_hbm, out_hbm,
                    g0_v, g1_v, i0_v, i1_v, w0_v, w1_v, sem0, sem1):
        wid = lax.axis_index("s") * 2 + lax.axis_index("c")
        base = wid * CH
        pltpu.sync_copy(w0_hbm.at[pl.ds(base, CH)], w0_v)
        pltpu.sync_copy(w1_hbm.at[pl.ds(base, CH)], w1_v)
        for sc in range(CH // SUB):
            sbase = base + sc * SUB
            pltpu.sync_copy(pos0_hbm.at[pl.ds(sbase, SUB)], i0_v)
            pltpu.sync_copy(pos1_hbm.at[pl.ds(sbase, SUB)], i1_v)
            cp0 = pltpu.async_copy(ys_hbm.at[i0_v], g0_v, sem0)
            cp1 = pltpu.async_copy(ys_hbm.at[i1_v], g1_v, sem1)
            cp0.wait()
            cp1.wait()
            ws = []
            for r in range(SUB):
                lane = (sc * SUB + r) % 16
                chunk = (sc * SUB + r) - lane
                ws.append((w0_v[pl.ds(chunk, 16)][lane],
                           w1_v[pl.ds(chunk, 16)][lane]))

            def col(j, carry):
                for r in range(SUB):
                    a, bw = ws[r]
                    v0 = g0_v[r, pl.ds(j * 16, 16)]
                    v1 = g1_v[r, pl.ds(j * 16, 16)]
                    g0_v[r, pl.ds(j * 16, 16)] = v0 * a + v1 * bw
                return carry

            lax.fori_loop(0, D // 16, col, 0)
            pltpu.sync_copy(g0_v, out_hbm.at[pl.ds(sbase, SUB)])

    return _sc_combine


def kernel(x, W_router, W_gate, W_up, W_down):
    pos0, pos1, w0, w1, bexp, bvalid = _router_call()(x, W_router)
    pos0 = pos0.reshape(T)
    pos1 = pos1.reshape(T)
    xs = _make_sc_scatter()(x, pos0, pos1)
    ys = _gmm_call()(bexp.reshape(NB), bvalid.reshape(NB), xs, W_gate, W_up,
                     W_down)
    out = _make_sc_combine()(ys, pos0, pos1, w0.reshape(T), w1.reshape(T))
    return out


# speedup vs baseline: 1.5710x; 1.0318x over previous
import functools

import jax
import jax.numpy as jnp
from jax import lax
from jax.experimental import pallas as pl
from jax.experimental.pallas import tpu as pltpu
from jax.experimental.pallas import tpu_sc as plsc

T = 2048
D = 1024
E = 8
F = 4096
TOPK = 2

F_TILE = 1024
NF = F // F_TILE
TB = 256
NB = 24
PADN = NB * TB

NW = 32
CH = T // NW
SUB = 32


def _router_body(x_ref, wr_ref, pos0_ref, pos1_ref, w0_ref, w1_ref,
                 bexp_ref, bvalid_ref, xp_ref):
    xbf = x_ref[...].astype(jnp.bfloat16)
    lo = lax.bitcast_convert_type(xbf[:, :D // 2], jnp.uint16)
    hi = lax.bitcast_convert_type(xbf[:, D // 2:], jnp.uint16)
    xp = lo.astype(jnp.uint32) | (hi.astype(jnp.uint32) << 16)
    xp_ref[...] = lax.bitcast_convert_type(xp, jnp.int32)

    logits = jnp.dot(x_ref[...], wr_ref[...], preferred_element_type=jnp.float32)
    eidx = lax.broadcasted_iota(jnp.int32, (T, E), 1)
    m1 = jnp.max(logits, axis=-1, keepdims=True)
    e0 = jnp.min(jnp.where(logits == m1, eidx, E), axis=-1, keepdims=True)
    l2 = jnp.where(eidx == e0, -jnp.inf, logits)
    m2 = jnp.max(l2, axis=-1, keepdims=True)
    e1 = jnp.min(jnp.where(l2 == m2, eidx, E), axis=-1, keepdims=True)
    w0 = 1.0 / (1.0 + jnp.exp(m2 - m1))
    w0_ref[...] = w0
    w1_ref[...] = 1.0 - w0

    oh0 = (eidx == e0).astype(jnp.float32)
    oh1 = (eidx == e1).astype(jnp.float32)

    CB = 256
    r = lax.broadcasted_iota(jnp.int32, (CB, CB), 0)
    c = lax.broadcasted_iota(jnp.int32, (CB, CB), 1)
    ltri = (r >= c).astype(jnp.float32)

    def ranks(oh):
        tot = jnp.zeros((1, E), jnp.float32)
        parts = []
        for b in range(T // CB):
            blk = oh[b * CB:(b + 1) * CB, :]
            cum = jnp.dot(ltri, blk, preferred_element_type=jnp.float32) + tot
            parts.append(jnp.sum(cum * blk, axis=1, keepdims=True) - 1.0)
            tot = tot + jnp.sum(blk, axis=0, keepdims=True)
        return jnp.concatenate(parts, axis=0), tot

    rank0, cnt0 = ranks(oh0)
    rank1, cnt1 = ranks(oh1)
    cnt = cnt0 + cnt1
    nblk = jnp.floor((cnt + (TB - 1)) / TB)
    stri = (lax.broadcasted_iota(jnp.int32, (E, E), 0)
            < lax.broadcasted_iota(jnp.int32, (E, E), 1)).astype(jnp.float32)
    offblk = jnp.dot(nblk, stri, preferred_element_type=jnp.float32)
    off = offblk * TB

    off0 = jnp.sum(oh0 * off, axis=1, keepdims=True)
    off1 = jnp.sum(oh1 * off, axis=1, keepdims=True)
    c0at1 = jnp.sum(oh1 * cnt0, axis=1, keepdims=True)
    pos0_ref[...] = (off0 + rank0).astype(jnp.int32)
    pos1_ref[...] = (off1 + c0at1 + rank1).astype(jnp.int32)

    bidx = lax.broadcasted_iota(jnp.int32, (NB, E), 0).astype(jnp.float32)
    bexp = jnp.sum((bidx >= offblk).astype(jnp.float32), axis=1,
                   keepdims=True) - 1.0
    bexp = jnp.clip(bexp, 0.0, E - 1.0)
    ohb = (lax.broadcasted_iota(jnp.int32, (NB, E), 1).astype(jnp.float32)
           == bexp)
    offb = jnp.sum(jnp.where(ohb, off, 0.0), axis=1, keepdims=True)
    cntb = jnp.sum(jnp.where(ohb, cnt, 0.0), axis=1, keepdims=True)
    brow = lax.broadcasted_iota(jnp.int32, (NB, 1), 0).astype(jnp.float32) * TB
    bexp_ref[...] = bexp.astype(jnp.int32)
    bvalid_ref[...] = ((brow - offb) < cntb).astype(jnp.int32)


def _router_call():
    return pl.pallas_call(
        _router_body,
        out_shape=(
            jax.ShapeDtypeStruct((T, 1), jnp.int32),
            jax.ShapeDtypeStruct((T, 1), jnp.int32),
            jax.ShapeDtypeStruct((T, 1), jnp.float32),
            jax.ShapeDtypeStruct((T, 1), jnp.float32),
            jax.ShapeDtypeStruct((NB, 1), jnp.int32),
            jax.ShapeDtypeStruct((NB, 1), jnp.int32),
            jax.ShapeDtypeStruct((T, D // 2), jnp.int32),
        ),
    )


@functools.lru_cache(maxsize=None)
def _make_sc_scatter():
    mesh = plsc.VectorSubcoreMesh(core_axis_name="c", subcore_axis_name="s")

    @functools.partial(
        pl.kernel,
        mesh=mesh,
        out_type=jax.ShapeDtypeStruct((PADN, D // 2), jnp.int32),
        scratch_types=[
            pltpu.VMEM((CH, D // 2), jnp.int32),
            pltpu.VMEM((CH,), jnp.int32),
            pltpu.VMEM((CH,), jnp.int32),
            pltpu.SemaphoreType.DMA,
            pltpu.SemaphoreType.DMA,
        ],
    )
    def _sc_scatter(x_hbm, pos0_hbm, pos1_hbm, xs_hbm, rows_v, idx0_v, idx1_v,
                    sem0, sem1):
        wid = lax.axis_index("s") * 2 + lax.axis_index("c")
        base = wid * CH
        pltpu.sync_copy(x_hbm.at[pl.ds(base, CH)], rows_v)
        pltpu.sync_copy(pos0_hbm.at[pl.ds(base, CH)], idx0_v)
        pltpu.sync_copy(pos1_hbm.at[pl.ds(base, CH)], idx1_v)
        cp0 = pltpu.async_copy(rows_v, xs_hbm.at[idx0_v], sem0)
        cp1 = pltpu.async_copy(rows_v, xs_hbm.at[idx1_v], sem1)
        cp0.wait()
        cp1.wait()

    return _sc_scatter


def _gmm_body(bexp_sm, bvalid_sm, xs_ref, wg_ref, wu_ref, wd_ref, out_ref,
              acc_ref):
    f = pl.program_id(0)
    b = pl.program_id(1)

    @pl.when(bvalid_sm[b] == 1)
    def _():
        xiu = lax.bitcast_convert_type(xs_ref[...], jnp.uint32)
        xlo = lax.bitcast_convert_type(
            (xiu & 0xFFFF).astype(jnp.uint16), jnp.bfloat16)
        xhi = lax.bitcast_convert_type(
            (xiu >> 16).astype(jnp.uint16), jnp.bfloat16)
        xb = jnp.concatenate([xlo, xhi], axis=1)
        g = jnp.dot(xb, wg_ref[0].astype(jnp.bfloat16),
                    preferred_element_type=jnp.float32)
        u = jnp.dot(xb, wu_ref[0].astype(jnp.bfloat16),
                    preferred_element_type=jnp.float32)
        h = (g * jax.nn.sigmoid(g)) * u
        y = jnp.dot(h.astype(jnp.bfloat16), wd_ref[0].astype(jnp.bfloat16),
                    preferred_element_type=jnp.float32)

        @pl.when(f == 0)
        def _():
            acc_ref[pl.ds(b * TB, TB), :] = y

        @pl.when(f != 0)
        def _():
            acc_ref[pl.ds(b * TB, TB), :] += y

    @pl.when(f == NF - 1)
    def _():
        out_ref[...] = acc_ref[pl.ds(b * TB, TB), :]


def _gmm_call():
    return pl.pallas_call(
        _gmm_body,
        grid_spec=pltpu.PrefetchScalarGridSpec(
            num_scalar_prefetch=2,
            grid=(NF, NB),
            in_specs=[
                pl.BlockSpec((TB, D // 2), lambda f, b, be, bv: (b, 0)),
                pl.BlockSpec((1, D, F_TILE),
                             lambda f, b, be, bv: (be[b], 0, f)),
                pl.BlockSpec((1, D, F_TILE),
                             lambda f, b, be, bv: (be[b], 0, f)),
                pl.BlockSpec((1, F_TILE, D),
                             lambda f, b, be, bv: (be[b], f, 0)),
            ],
            out_specs=pl.BlockSpec(
                (TB, D),
                lambda f, b, be, bv: (jnp.where(f == NF - 1, b, 0), 0)),
            scratch_shapes=[pltpu.VMEM((PADN, D), jnp.float32)],
        ),
        out_shape=jax.ShapeDtypeStruct((PADN, D), jnp.float32),
    )


@functools.lru_cache(maxsize=None)
def _make_sc_combine():
    mesh = plsc.VectorSubcoreMesh(core_axis_name="c", subcore_axis_name="s")

    @functools.partial(
        pl.kernel,
        mesh=mesh,
        out_type=jax.ShapeDtypeStruct((T, D), jnp.float32),
        scratch_types=[
            pltpu.VMEM((SUB, D), jnp.float32),
            pltpu.VMEM((SUB, D), jnp.float32),
            pltpu.VMEM((SUB,), jnp.int32),
            pltpu.VMEM((SUB,), jnp.int32),
            pltpu.VMEM((CH,), jnp.float32),
            pltpu.VMEM((CH,), jnp.float32),
            pltpu.SemaphoreType.DMA,
            pltpu.SemaphoreType.DMA,
        ],
    )
    def _sc_combine(ys_hbm, pos0_hbm, pos1_hbm, w0_hbm, w1_hbm, out_hbm,
                    g0_v, g1_v, i0_v, i1_v, w0_v, w1_v, sem0, sem1):
        wid = lax.axis_index("s") * 2 + lax.axis_index("c")
        base = wid * CH
        pltpu.sync_copy(w0_hbm.at[pl.ds(base, CH)], w0_v)
        pltpu.sync_copy(w1_hbm.at[pl.ds(base, CH)], w1_v)
        for sc in range(CH // SUB):
            sbase = base + sc * SUB
            pltpu.sync_copy(pos0_hbm.at[pl.ds(sbase, SUB)], i0_v)
            pltpu.sync_copy(pos1_hbm.at[pl.ds(sbase, SUB)], i1_v)
            cp0 = pltpu.async_copy(ys_hbm.at[i0_v], g0_v, sem0)
            cp1 = pltpu.async_copy(ys_hbm.at[i1_v], g1_v, sem1)
            cp0.wait()
            cp1.wait()
            ws = []
            for r in range(SUB):
                lane = (sc * SUB + r) % 16
                chunk = (sc * SUB + r) - lane
                ws.append((w0_v[pl.ds(chunk, 16)][lane],
                           w1_v[pl.ds(chunk, 16)][lane]))

            def col(j, carry):
                for r in range(SUB):
                    a, bw = ws[r]
                    v0 = g0_v[r, pl.ds(j * 16, 16)]
                    v1 = g1_v[r, pl.ds(j * 16, 16)]
                    g0_v[r, pl.ds(j * 16, 16)] = v0 * a + v1 * bw
                return carry

            lax.fori_loop(0, D // 16, col, 0)
            pltpu.sync_copy(g0_v, out_hbm.at[pl.ds(sbase, SUB)])

    return _sc_combine


def kernel(x, W_router, W_gate, W_up, W_down):
    pos0, pos1, w0, w1, bexp, bvalid, xp = _router_call()(x, W_router)
    pos0 = pos0.reshape(T)
    pos1 = pos1.reshape(T)
    xs = _make_sc_scatter()(xp, pos0, pos1)
    ys = _gmm_call()(bexp.reshape(NB), bvalid.reshape(NB), xs, W_gate, W_up,
                     W_down)
    out = _make_sc_combine()(ys, pos0, pos1, w0.reshape(T), w1.reshape(T))
    return out


# speedup vs baseline: 1.6057x; 1.0221x over previous
import functools

import jax
import jax.numpy as jnp
from jax import lax
from jax.experimental import pallas as pl
from jax.experimental.pallas import tpu as pltpu
from jax.experimental.pallas import tpu_sc as plsc

T = 2048
D = 1024
E = 8
F = 4096
TOPK = 2

F_TILE = 1024
NF = F // F_TILE
TB = 256
NB = 24
PADN = NB * TB

NW = 32
CH = T // NW
SUB = 32


def _router_body(x_ref, wr_ref, pos0_ref, pos1_ref, w0_ref, w1_ref,
                 bexp_ref, bvalid_ref, xp_ref):
    xbf = x_ref[...].astype(jnp.bfloat16)
    lo = lax.bitcast_convert_type(xbf[:, :D // 2], jnp.uint16)
    hi = lax.bitcast_convert_type(xbf[:, D // 2:], jnp.uint16)
    xp = lo.astype(jnp.uint32) | (hi.astype(jnp.uint32) << 16)
    xp_ref[...] = lax.bitcast_convert_type(xp, jnp.int32)

    logits = jnp.dot(x_ref[...], wr_ref[...], preferred_element_type=jnp.float32)
    eidx = lax.broadcasted_iota(jnp.int32, (T, E), 1)
    m1 = jnp.max(logits, axis=-1, keepdims=True)
    e0 = jnp.min(jnp.where(logits == m1, eidx, E), axis=-1, keepdims=True)
    l2 = jnp.where(eidx == e0, -jnp.inf, logits)
    m2 = jnp.max(l2, axis=-1, keepdims=True)
    e1 = jnp.min(jnp.where(l2 == m2, eidx, E), axis=-1, keepdims=True)
    w0 = 1.0 / (1.0 + jnp.exp(m2 - m1))
    w0_ref[...] = w0
    w1_ref[...] = 1.0 - w0

    oh0 = (eidx == e0).astype(jnp.float32)
    oh1 = (eidx == e1).astype(jnp.float32)

    CB = 256
    r = lax.broadcasted_iota(jnp.int32, (CB, CB), 0)
    c = lax.broadcasted_iota(jnp.int32, (CB, CB), 1)
    ltri = (r >= c).astype(jnp.float32)

    def ranks(oh):
        tot = jnp.zeros((1, E), jnp.float32)
        parts = []
        for b in range(T // CB):
            blk = oh[b * CB:(b + 1) * CB, :]
            cum = jnp.dot(ltri, blk, preferred_element_type=jnp.float32) + tot
            parts.append(jnp.sum(cum * blk, axis=1, keepdims=True) - 1.0)
            tot = tot + jnp.sum(blk, axis=0, keepdims=True)
        return jnp.concatenate(parts, axis=0), tot

    rank0, cnt0 = ranks(oh0)
    rank1, cnt1 = ranks(oh1)
    cnt = cnt0 + cnt1
    nblk = jnp.floor((cnt + (TB - 1)) / TB)
    stri = (lax.broadcasted_iota(jnp.int32, (E, E), 0)
            < lax.broadcasted_iota(jnp.int32, (E, E), 1)).astype(jnp.float32)
    offblk = jnp.dot(nblk, stri, preferred_element_type=jnp.float32)
    off = offblk * TB

    off0 = jnp.sum(oh0 * off, axis=1, keepdims=True)
    off1 = jnp.sum(oh1 * off, axis=1, keepdims=True)
    c0at1 = jnp.sum(oh1 * cnt0, axis=1, keepdims=True)
    pos0_ref[...] = (off0 + rank0).astype(jnp.int32)
    pos1_ref[...] = (off1 + c0at1 + rank1).astype(jnp.int32)

    bidx = lax.broadcasted_iota(jnp.int32, (NB, E), 0).astype(jnp.float32)
    bexp = jnp.sum((bidx >= offblk).astype(jnp.float32), axis=1,
                   keepdims=True) - 1.0
    bexp = jnp.clip(bexp, 0.0, E - 1.0)
    ohb = (lax.broadcasted_iota(jnp.int32, (NB, E), 1).astype(jnp.float32)
           == bexp)
    offb = jnp.sum(jnp.where(ohb, off, 0.0), axis=1, keepdims=True)
    cntb = jnp.sum(jnp.where(ohb, cnt, 0.0), axis=1, keepdims=True)
    brow = lax.broadcasted_iota(jnp.int32, (NB, 1), 0).astype(jnp.float32) * TB
    bexp_ref[...] = bexp.astype(jnp.int32)
    bvalid_ref[...] = ((brow - offb) < cntb).astype(jnp.int32)


def _router_call():
    return pl.pallas_call(
        _router_body,
        out_shape=(
            jax.ShapeDtypeStruct((T, 1), jnp.int32),
            jax.ShapeDtypeStruct((T, 1), jnp.int32),
            jax.ShapeDtypeStruct((T, 1), jnp.float32),
            jax.ShapeDtypeStruct((T, 1), jnp.float32),
            jax.ShapeDtypeStruct((NB, 1), jnp.int32),
            jax.ShapeDtypeStruct((NB, 1), jnp.int32),
            jax.ShapeDtypeStruct((T, D // 2), jnp.int32),
        ),
    )


@functools.lru_cache(maxsize=None)
def _make_sc_scatter():
    mesh = plsc.VectorSubcoreMesh(core_axis_name="c", subcore_axis_name="s")

    @functools.partial(
        pl.kernel,
        mesh=mesh,
        out_type=jax.ShapeDtypeStruct((PADN, D // 2), jnp.int32),
        scratch_types=[
            pltpu.VMEM((CH, D // 2), jnp.int32),
            pltpu.VMEM((CH,), jnp.int32),
            pltpu.VMEM((CH,), jnp.int32),
            pltpu.SemaphoreType.DMA,
            pltpu.SemaphoreType.DMA,
        ],
    )
    def _sc_scatter(x_hbm, pos0_hbm, pos1_hbm, xs_hbm, rows_v, idx0_v, idx1_v,
                    sem0, sem1):
        wid = lax.axis_index("s") * 2 + lax.axis_index("c")
        base = wid * CH
        pltpu.sync_copy(x_hbm.at[pl.ds(base, CH)], rows_v)
        pltpu.sync_copy(pos0_hbm.at[pl.ds(base, CH)], idx0_v)
        pltpu.sync_copy(pos1_hbm.at[pl.ds(base, CH)], idx1_v)
        cp0 = pltpu.async_copy(rows_v, xs_hbm.at[idx0_v], sem0)
        cp1 = pltpu.async_copy(rows_v, xs_hbm.at[idx1_v], sem1)
        cp0.wait()
        cp1.wait()

    return _sc_scatter


def _gmm_body(bexp_sm, bvalid_sm, xs_ref, wg_ref, wu_ref, wd_ref, out_ref,
              acc_ref):
    f = pl.program_id(0)
    b = pl.program_id(1)

    @pl.when(bvalid_sm[b] == 1)
    def _():
        xiu = lax.bitcast_convert_type(xs_ref[...], jnp.uint32)
        xlo = lax.bitcast_convert_type(
            (xiu & 0xFFFF).astype(jnp.uint16), jnp.bfloat16)
        xhi = lax.bitcast_convert_type(
            (xiu >> 16).astype(jnp.uint16), jnp.bfloat16)
        xb = jnp.concatenate([xlo, xhi], axis=1)
        g = jnp.dot(xb, wg_ref[0].astype(jnp.bfloat16),
                    preferred_element_type=jnp.float32)
        u = jnp.dot(xb, wu_ref[0].astype(jnp.bfloat16),
                    preferred_element_type=jnp.float32)
        h = (g * jax.nn.sigmoid(g)) * u
        y = jnp.dot(h.astype(jnp.bfloat16), wd_ref[0].astype(jnp.bfloat16),
                    preferred_element_type=jnp.float32)

        @pl.when(f == 0)
        def _():
            acc_ref[pl.ds(b * TB, TB), :] = y

        @pl.when(f != 0)
        def _():
            acc_ref[pl.ds(b * TB, TB), :] += y

    @pl.when(f == NF - 1)
    def _():
        rows = acc_ref[pl.ds(b * TB, TB), :]
        lo = lax.bitcast_convert_type(
            rows[:, :D // 2].astype(jnp.bfloat16), jnp.uint16)
        hi = lax.bitcast_convert_type(
            rows[:, D // 2:].astype(jnp.bfloat16), jnp.uint16)
        packed = lo.astype(jnp.uint32) | (hi.astype(jnp.uint32) << 16)
        out_ref[...] = lax.bitcast_convert_type(packed, jnp.int32)


def _gmm_call():
    return pl.pallas_call(
        _gmm_body,
        grid_spec=pltpu.PrefetchScalarGridSpec(
            num_scalar_prefetch=2,
            grid=(NF, NB),
            in_specs=[
                pl.BlockSpec((TB, D // 2), lambda f, b, be, bv: (b, 0)),
                pl.BlockSpec((1, D, F_TILE),
                             lambda f, b, be, bv: (be[b], 0, f)),
                pl.BlockSpec((1, D, F_TILE),
                             lambda f, b, be, bv: (be[b], 0, f)),
                pl.BlockSpec((1, F_TILE, D),
                             lambda f, b, be, bv: (be[b], f, 0)),
            ],
            out_specs=pl.BlockSpec(
                (TB, D // 2),
                lambda f, b, be, bv: (jnp.where(f == NF - 1, b, 0), 0)),
            scratch_shapes=[pltpu.VMEM((PADN, D), jnp.float32)],
        ),
        out_shape=jax.ShapeDtypeStruct((PADN, D // 2), jnp.int32),
    )


@functools.lru_cache(maxsize=None)
def _make_sc_combine():
    mesh = plsc.VectorSubcoreMesh(core_axis_name="c", subcore_axis_name="s")

    @functools.partial(
        pl.kernel,
        mesh=mesh,
        out_type=jax.ShapeDtypeStruct((T, D), jnp.float32),
        scratch_types=[
            pltpu.VMEM((SUB, D // 2), jnp.int32),
            pltpu.VMEM((SUB, D // 2), jnp.int32),
            pltpu.VMEM((SUB, D), jnp.float32),
            pltpu.VMEM((SUB,), jnp.int32),
            pltpu.VMEM((SUB,), jnp.int32),
            pltpu.VMEM((CH,), jnp.float32),
            pltpu.VMEM((CH,), jnp.float32),
            pltpu.SemaphoreType.DMA,
            pltpu.SemaphoreType.DMA,
        ],
    )
    def _sc_combine(ys_hbm, pos0_hbm, pos1_hbm, w0_hbm, w1_hbm, out_hbm,
                    g0_v, g1_v, o_v, i0_v, i1_v, w0_v, w1_v, sem0, sem1):
        wid = lax.axis_index("s") * 2 + lax.axis_index("c")
        base = wid * CH
        pltpu.sync_copy(w0_hbm.at[pl.ds(base, CH)], w0_v)
        pltpu.sync_copy(w1_hbm.at[pl.ds(base, CH)], w1_v)
        for sc in range(CH // SUB):
            sbase = base + sc * SUB
            pltpu.sync_copy(pos0_hbm.at[pl.ds(sbase, SUB)], i0_v)
            pltpu.sync_copy(pos1_hbm.at[pl.ds(sbase, SUB)], i1_v)
            cp0 = pltpu.async_copy(ys_hbm.at[i0_v], g0_v, sem0)
            cp1 = pltpu.async_copy(ys_hbm.at[i1_v], g1_v, sem1)
            cp0.wait()
            cp1.wait()
            ws = []
            for r in range(SUB):
                lane = (sc * SUB + r) % 16
                chunk = (sc * SUB + r) - lane
                ws.append((w0_v[pl.ds(chunk, 16)][lane],
                           w1_v[pl.ds(chunk, 16)][lane]))

            mask = jnp.int32(-65536)

            def col(j, carry):
                for r in range(SUB):
                    a, bw = ws[r]
                    u0 = g0_v[r, pl.ds(j * 16, 16)]
                    u1 = g1_v[r, pl.ds(j * 16, 16)]
                    lo0 = lax.bitcast_convert_type(u0 << 16, jnp.float32)
                    hi0 = lax.bitcast_convert_type(u0 & mask, jnp.float32)
                    lo1 = lax.bitcast_convert_type(u1 << 16, jnp.float32)
                    hi1 = lax.bitcast_convert_type(u1 & mask, jnp.float32)
                    o_v[r, pl.ds(j * 16, 16)] = lo0 * a + lo1 * bw
                    o_v[r, pl.ds(D // 2 + j * 16, 16)] = hi0 * a + hi1 * bw
                return carry

            lax.fori_loop(0, D // 2 // 16, col, 0)
            pltpu.sync_copy(o_v, out_hbm.at[pl.ds(sbase, SUB)])

    return _sc_combine


def kernel(x, W_router, W_gate, W_up, W_down):
    pos0, pos1, w0, w1, bexp, bvalid, xp = _router_call()(x, W_router)
    pos0 = pos0.reshape(T)
    pos1 = pos1.reshape(T)
    xs = _make_sc_scatter()(xp, pos0, pos1)
    ys = _gmm_call()(bexp.reshape(NB), bvalid.reshape(NB), xs, W_gate, W_up,
                     W_down)
    out = _make_sc_combine()(ys, pos0, pos1, w0.reshape(T), w1.reshape(T))
    return out


# speedup vs baseline: 1.6103x; 1.0028x over previous
import functools

import jax
import jax.numpy as jnp
from jax import lax
from jax.experimental import pallas as pl
from jax.experimental.pallas import tpu as pltpu
from jax.experimental.pallas import tpu_sc as plsc

T = 2048
D = 1024
E = 8
F = 4096
TOPK = 2

F_TILE = 1024
NF = F // F_TILE
TB = 256
NB = 24
PADN = NB * TB

NW = 32
CH = T // NW
SUB = 32


def _router_body(x_ref, wr_ref, pos0_ref, pos1_ref, w0_ref, w1_ref,
                 bexp_ref, bvalid_ref, xp_ref):
    xbf = x_ref[...].astype(jnp.bfloat16)
    lo = lax.bitcast_convert_type(xbf[:, :D // 2], jnp.uint16)
    hi = lax.bitcast_convert_type(xbf[:, D // 2:], jnp.uint16)
    xp = lo.astype(jnp.uint32) | (hi.astype(jnp.uint32) << 16)
    xp_ref[...] = lax.bitcast_convert_type(xp, jnp.int32)

    logits = jnp.dot(x_ref[...], wr_ref[...], preferred_element_type=jnp.float32)
    eidx = lax.broadcasted_iota(jnp.int32, (T, E), 1)
    m1 = jnp.max(logits, axis=-1, keepdims=True)
    e0 = jnp.min(jnp.where(logits == m1, eidx, E), axis=-1, keepdims=True)
    l2 = jnp.where(eidx == e0, -jnp.inf, logits)
    m2 = jnp.max(l2, axis=-1, keepdims=True)
    e1 = jnp.min(jnp.where(l2 == m2, eidx, E), axis=-1, keepdims=True)
    w0 = 1.0 / (1.0 + jnp.exp(m2 - m1))
    w0_ref[...] = w0
    w1_ref[...] = 1.0 - w0

    oh0 = (eidx == e0).astype(jnp.float32)
    oh1 = (eidx == e1).astype(jnp.float32)

    CB = 256
    r = lax.broadcasted_iota(jnp.int32, (CB, CB), 0)
    c = lax.broadcasted_iota(jnp.int32, (CB, CB), 1)
    ltri = (r >= c).astype(jnp.float32)

    def ranks(oh):
        tot = jnp.zeros((1, E), jnp.float32)
        parts = []
        for b in range(T // CB):
            blk = oh[b * CB:(b + 1) * CB, :]
            cum = jnp.dot(ltri, blk, preferred_element_type=jnp.float32) + tot
            parts.append(jnp.sum(cum * blk, axis=1, keepdims=True) - 1.0)
            tot = tot + jnp.sum(blk, axis=0, keepdims=True)
        return jnp.concatenate(parts, axis=0), tot

    rank0, cnt0 = ranks(oh0)
    rank1, cnt1 = ranks(oh1)
    cnt = cnt0 + cnt1
    nblk = jnp.floor((cnt + (TB - 1)) / TB)
    stri = (lax.broadcasted_iota(jnp.int32, (E, E), 0)
            < lax.broadcasted_iota(jnp.int32, (E, E), 1)).astype(jnp.float32)
    offblk = jnp.dot(nblk, stri, preferred_element_type=jnp.float32)
    off = offblk * TB

    off0 = jnp.sum(oh0 * off, axis=1, keepdims=True)
    off1 = jnp.sum(oh1 * off, axis=1, keepdims=True)
    c0at1 = jnp.sum(oh1 * cnt0, axis=1, keepdims=True)
    pos0_ref[...] = (off0 + rank0).astype(jnp.int32)
    pos1_ref[...] = (off1 + c0at1 + rank1).astype(jnp.int32)

    bidx = lax.broadcasted_iota(jnp.int32, (NB, E), 0).astype(jnp.float32)
    bexp = jnp.sum((bidx >= offblk).astype(jnp.float32), axis=1,
                   keepdims=True) - 1.0
    bexp = jnp.clip(bexp, 0.0, E - 1.0)
    ohb = (lax.broadcasted_iota(jnp.int32, (NB, E), 1).astype(jnp.float32)
           == bexp)
    offb = jnp.sum(jnp.where(ohb, off, 0.0), axis=1, keepdims=True)
    cntb = jnp.sum(jnp.where(ohb, cnt, 0.0), axis=1, keepdims=True)
    brow = lax.broadcasted_iota(jnp.int32, (NB, 1), 0).astype(jnp.float32) * TB
    bexp_ref[...] = bexp.astype(jnp.int32)
    bvalid_ref[...] = ((brow - offb) < cntb).astype(jnp.int32)


def _router_call():
    return pl.pallas_call(
        _router_body,
        out_shape=(
            jax.ShapeDtypeStruct((T, 1), jnp.int32),
            jax.ShapeDtypeStruct((T, 1), jnp.int32),
            jax.ShapeDtypeStruct((T, 1), jnp.float32),
            jax.ShapeDtypeStruct((T, 1), jnp.float32),
            jax.ShapeDtypeStruct((NB, 1), jnp.int32),
            jax.ShapeDtypeStruct((NB, 1), jnp.int32),
            jax.ShapeDtypeStruct((T, D // 2), jnp.int32),
        ),
    )


@functools.lru_cache(maxsize=None)
def _make_sc_scatter():
    mesh = plsc.VectorSubcoreMesh(core_axis_name="c", subcore_axis_name="s")

    @functools.partial(
        pl.kernel,
        mesh=mesh,
        out_type=jax.ShapeDtypeStruct((PADN, D // 2), jnp.int32),
        scratch_types=[
            pltpu.VMEM((CH, D // 2), jnp.int32),
            pltpu.VMEM((CH,), jnp.int32),
            pltpu.VMEM((CH,), jnp.int32),
            pltpu.SemaphoreType.DMA,
            pltpu.SemaphoreType.DMA,
        ],
    )
    def _sc_scatter(x_hbm, pos0_hbm, pos1_hbm, xs_hbm, rows_v, idx0_v, idx1_v,
                    sem0, sem1):
        wid = lax.axis_index("s") * 2 + lax.axis_index("c")
        base = wid * CH
        pltpu.sync_copy(x_hbm.at[pl.ds(base, CH)], rows_v)
        pltpu.sync_copy(pos0_hbm.at[pl.ds(base, CH)], idx0_v)
        pltpu.sync_copy(pos1_hbm.at[pl.ds(base, CH)], idx1_v)
        cp0 = pltpu.async_copy(rows_v, xs_hbm.at[idx0_v], sem0)
        cp1 = pltpu.async_copy(rows_v, xs_hbm.at[idx1_v], sem1)
        cp0.wait()
        cp1.wait()

    return _sc_scatter


def _gmm_body(bexp_sm, bvalid_sm, xs_ref, wg_ref, wu_ref, wd_ref, out_ref,
              acc_ref):
    f = pl.program_id(0)
    b = pl.program_id(1)

    @pl.when(bvalid_sm[b] == 1)
    def _():
        xiu = lax.bitcast_convert_type(xs_ref[...], jnp.uint32)
        xlo = lax.bitcast_convert_type(
            (xiu & 0xFFFF).astype(jnp.uint16), jnp.bfloat16)
        xhi = lax.bitcast_convert_type(
            (xiu >> 16).astype(jnp.uint16), jnp.bfloat16)
        xb = jnp.concatenate([xlo, xhi], axis=1)
        g = jnp.dot(xb, wg_ref[0].astype(jnp.bfloat16),
                    preferred_element_type=jnp.float32)
        u = jnp.dot(xb, wu_ref[0].astype(jnp.bfloat16),
                    preferred_element_type=jnp.float32)
        h = (g * jax.nn.sigmoid(g)) * u
        y = jnp.dot(h.astype(jnp.bfloat16), wd_ref[0].astype(jnp.bfloat16),
                    preferred_element_type=jnp.float32)

        @pl.when(f == 0)
        def _():
            acc_ref[pl.ds(b * TB, TB), :] = y

        @pl.when(f != 0)
        def _():
            acc_ref[pl.ds(b * TB, TB), :] += y

    @pl.when(f == NF - 1)
    def _():
        rows = acc_ref[pl.ds(b * TB, TB), :]
        lo = lax.bitcast_convert_type(
            rows[:, :D // 2].astype(jnp.bfloat16), jnp.uint16)
        hi = lax.bitcast_convert_type(
            rows[:, D // 2:].astype(jnp.bfloat16), jnp.uint16)
        packed = lo.astype(jnp.uint32) | (hi.astype(jnp.uint32) << 16)
        out_ref[...] = lax.bitcast_convert_type(packed, jnp.int32)


def _gmm_call():
    return pl.pallas_call(
        _gmm_body,
        grid_spec=pltpu.PrefetchScalarGridSpec(
            num_scalar_prefetch=2,
            grid=(NF, NB),
            in_specs=[
                pl.BlockSpec((TB, D // 2), lambda f, b, be, bv: (b, 0)),
                pl.BlockSpec((1, D, F_TILE),
                             lambda f, b, be, bv: (be[b], 0, f)),
                pl.BlockSpec((1, D, F_TILE),
                             lambda f, b, be, bv: (be[b], 0, f)),
                pl.BlockSpec((1, F_TILE, D),
                             lambda f, b, be, bv: (be[b], f, 0)),
            ],
            out_specs=pl.BlockSpec(
                (TB, D // 2),
                lambda f, b, be, bv: (jnp.where(f == NF - 1, b, 0), 0)),
            scratch_shapes=[pltpu.VMEM((PADN, D), jnp.float32)],
        ),
        out_shape=jax.ShapeDtypeStruct((PADN, D // 2), jnp.int32),
    )


@functools.lru_cache(maxsize=None)
def _make_sc_combine():
    mesh = plsc.VectorSubcoreMesh(core_axis_name="c", subcore_axis_name="s")

    @functools.partial(
        pl.kernel,
        mesh=mesh,
        out_type=jax.ShapeDtypeStruct((T, D), jnp.float32),
        scratch_types=[
            pltpu.VMEM((2, SUB, D // 2), jnp.int32),
            pltpu.VMEM((2, SUB, D // 2), jnp.int32),
            pltpu.VMEM((SUB, D), jnp.float32),
            pltpu.VMEM((2, SUB), jnp.int32),
            pltpu.VMEM((2, SUB), jnp.int32),
            pltpu.VMEM((CH,), jnp.float32),
            pltpu.VMEM((CH,), jnp.float32),
            pltpu.SemaphoreType.DMA,
            pltpu.SemaphoreType.DMA,
            pltpu.SemaphoreType.DMA,
            pltpu.SemaphoreType.DMA,
        ],
    )
    def _sc_combine(ys_hbm, pos0_hbm, pos1_hbm, w0_hbm, w1_hbm, out_hbm,
                    g0_v, g1_v, o_v, i0_v, i1_v, w0_v, w1_v,
                    sem0, sem1, sem2, sem3):
        wid = lax.axis_index("s") * 2 + lax.axis_index("c")
        base = wid * CH
        pltpu.sync_copy(w0_hbm.at[pl.ds(base, CH)], w0_v)
        pltpu.sync_copy(w1_hbm.at[pl.ds(base, CH)], w1_v)
        pltpu.sync_copy(pos0_hbm.at[pl.ds(base, SUB)], i0_v.at[0])
        pltpu.sync_copy(pos1_hbm.at[pl.ds(base, SUB)], i1_v.at[0])
        pltpu.sync_copy(pos0_hbm.at[pl.ds(base + SUB, SUB)], i0_v.at[1])
        pltpu.sync_copy(pos1_hbm.at[pl.ds(base + SUB, SUB)], i1_v.at[1])
        cps = [
            pltpu.async_copy(ys_hbm.at[i0_v.at[0]], g0_v.at[0], sem0),
            pltpu.async_copy(ys_hbm.at[i1_v.at[0]], g1_v.at[0], sem1),
            pltpu.async_copy(ys_hbm.at[i0_v.at[1]], g0_v.at[1], sem2),
            pltpu.async_copy(ys_hbm.at[i1_v.at[1]], g1_v.at[1], sem3),
        ]
        for sc in range(CH // SUB):
            sbase = base + sc * SUB
            cps[2 * sc].wait()
            cps[2 * sc + 1].wait()
            ws = []
            for r in range(SUB):
                lane = (sc * SUB + r) % 16
                chunk = (sc * SUB + r) - lane
                ws.append((w0_v[pl.ds(chunk, 16)][lane],
                           w1_v[pl.ds(chunk, 16)][lane]))

            mask = jnp.int32(-65536)

            def col(j, carry):
                for r in range(SUB):
                    a, bw = ws[r]
                    u0 = g0_v[sc, r, pl.ds(j * 16, 16)]
                    u1 = g1_v[sc, r, pl.ds(j * 16, 16)]
                    lo0 = lax.bitcast_convert_type(u0 << 16, jnp.float32)
                    hi0 = lax.bitcast_convert_type(u0 & mask, jnp.float32)
                    lo1 = lax.bitcast_convert_type(u1 << 16, jnp.float32)
                    hi1 = lax.bitcast_convert_type(u1 & mask, jnp.float32)
                    o_v[r, pl.ds(j * 16, 16)] = lo0 * a + lo1 * bw
                    o_v[r, pl.ds(D // 2 + j * 16, 16)] = hi0 * a + hi1 * bw
                return carry

            lax.fori_loop(0, D // 2 // 16, col, 0)
            pltpu.sync_copy(o_v, out_hbm.at[pl.ds(sbase, SUB)])

    return _sc_combine


def kernel(x, W_router, W_gate, W_up, W_down):
    pos0, pos1, w0, w1, bexp, bvalid, xp = _router_call()(x, W_router)
    pos0 = pos0.reshape(T)
    pos1 = pos1.reshape(T)
    xs = _make_sc_scatter()(xp, pos0, pos1)
    ys = _gmm_call()(bexp.reshape(NB), bvalid.reshape(NB), xs, W_gate, W_up,
                     W_down)
    out = _make_sc_combine()(ys, pos0, pos1, w0.reshape(T), w1.reshape(T))
    return out
